# baseline (speedup 1.0000x reference)
import functools

import jax
import jax.numpy as jnp
from jax import lax
from jax.experimental import pallas as pl
from jax.experimental.pallas import tpu as pltpu
from jax.experimental.pallas import tpu_sc as plsc

_D = 16
_BATCH = 16384
_FIELDS = 26
_B = _BATCH * _FIELDS
_NC = 2
_NS = 16
_NW = _NC * _NS
_BPW = _B // _NW
_NCHUNK = 8
_CHUNK = _BPW // _NCHUNK

_mesh = plsc.VectorSubcoreMesh(core_axis_name="c", subcore_axis_name="s")


@functools.partial(
    pl.kernel,
    mesh=_mesh,
    out_type=jax.ShapeDtypeStruct((_B, _D), jnp.float32),
    compiler_params=pltpu.CompilerParams(use_tc_tiling_on_sc=False),
    scratch_types=[
        pltpu.VMEM((_BPW,), jnp.int32),
        pltpu.VMEM((2, _CHUNK, _D), jnp.float32),
        pltpu.SemaphoreType.DMA,
        pltpu.SemaphoreType.DMA,
        pltpu.SemaphoreType.DMA,
        pltpu.SemaphoreType.DMA,
    ],
)
def _embed_gather(idx_hbm, table_hbm, out_hbm, idx_v, rows_v, g0, g1, o0, o1):
    wid = lax.axis_index("s") * _NC + lax.axis_index("c")
    base = wid * _BPW
    pltpu.sync_copy(idx_hbm.at[pl.ds(base, _BPW)], idx_v)

    gsem = (g0, g1)
    osem = (o0, o1)

    def start_gather(j):
        return pltpu.async_copy(
            table_hbm.at[idx_v.at[pl.ds(j * _CHUNK, _CHUNK)]],
            rows_v.at[j % 2], gsem[j % 2])

    def start_put(j):
        return pltpu.async_copy(
            rows_v.at[j % 2], out_hbm.at[pl.ds(base + j * _CHUNK, _CHUNK)],
            osem[j % 2])

    gathers = [None, None]
    puts = [None, None]
    gathers[0] = start_gather(0)
    for j in range(_NCHUNK):
        nb = (j + 1) % 2
        if j + 1 < _NCHUNK:
            if puts[nb] is not None:
                puts[nb].wait()
            gathers[nb] = start_gather(j + 1)
        gathers[j % 2].wait()
        puts[j % 2] = start_put(j)
    puts[(_NCHUNK - 1) % 2].wait()
    puts[_NCHUNK % 2].wait()


def kernel(e, table):
    idx = e.reshape(_B).astype(jnp.int32)
    out = _embed_gather(idx, table)
    return out.reshape(_BATCH, _FIELDS, _D)


# speedup vs baseline: 1.5257x; 1.5257x over previous
import functools

import jax
import jax.numpy as jnp
from jax import lax
from jax.experimental import pallas as pl
from jax.experimental.pallas import tpu as pltpu
from jax.experimental.pallas import tpu_sc as plsc

_D = 16
_BATCH = 16384
_FIELDS = 26
_B = _BATCH * _FIELDS
_NW = 32
_BPW = _B // _NW
_BB = _BATCH // _NW
_CB = 64
_NCHUNK = _BB // _CB
_CI = _CB * _FIELDS

_mesh = plsc.VectorSubcoreMesh(core_axis_name="c", subcore_axis_name="s")


@functools.partial(
    pl.kernel,
    mesh=_mesh,
    out_type=jax.ShapeDtypeStruct((_FIELDS, 2, _BATCH // 128, 8, 128),
                                  jnp.float32),
    compiler_params=pltpu.CompilerParams(use_tc_tiling_on_sc=False,
                                         needs_layout_passes=False),
    scratch_types=[
        pltpu.VMEM((_BPW,), jnp.int32),
        pltpu.VMEM((_CI, _D), jnp.float32),
        pltpu.VMEM((2, _FIELDS * 2, 8, _CB), jnp.float32),
        pltpu.SemaphoreType.DMA,
        pltpu.SemaphoreType.DMA,
        pltpu.SemaphoreType.DMA,
    ],
)
def _embed_gather(idx_hbm, table_hbm, out_hbm, idx_v, rows_v, planes_v,
                  gsem, o0, o1):
    wid = lax.axis_index("s") * 2 + lax.axis_index("c")
    base = wid * _BPW
    pltpu.sync_copy(idx_hbm.at[pl.ds(base, _BPW)], idx_v)
    osem = (o0, o1)

    puts = [[], []]
    for c in range(_NCHUNK):
        pltpu.async_copy(
            table_hbm.at[idx_v.at[pl.ds(c * _CI, _CI)]], rows_v, gsem,
        ).wait()

        buf = c % 2
        for h in puts[buf]:
            h.wait()
        puts[buf] = []
        pbuf = planes_v.at[buf]

        def body(b, _):
            iota16 = lax.iota(jnp.int32, 16)
            bvec = jnp.full((16,), 0, jnp.int32) + b
            for f in range(_FIELDS):
                row = rows_v[(b * _FIELDS + f), :]
                plsc.store_scatter(
                    pbuf,
                    [f * 2 + lax.shift_right_logical(iota16, 3),
                     lax.bitwise_and(iota16, 7), bvec],
                    row)
            return 0

        lax.fori_loop(0, _CB, body, 0)

        tc = wid * (_BB // 128) + c // 2
        l0 = (c % 2) * _CB
        for f in range(_FIELDS):
            for tr in range(2):
                h = pltpu.async_copy(
                    pbuf.at[f * 2 + tr],
                    out_hbm.at[f, tr, tc, :, pl.ds(l0, _CB)],
                    osem[buf])
                puts[buf].append(h)
    for hs in puts:
        for h in hs:
            h.wait()


def kernel(e, table):
    idx = e.reshape(_B).astype(jnp.int32)
    out5 = _embed_gather(idx, table)
    return out5.transpose(2, 4, 0, 1, 3).reshape(_BATCH, _FIELDS, _D)


# speedup vs baseline: 1.5417x; 1.0105x over previous
import functools

import jax
import jax.numpy as jnp
from jax import lax
from jax.experimental import pallas as pl
from jax.experimental.pallas import tpu as pltpu
from jax.experimental.pallas import tpu_sc as plsc

_D = 16
_BATCH = 16384
_FIELDS = 26
_B = _BATCH * _FIELDS
_NW = 32
_BPW = _B // _NW
_BB = _BATCH // _NW
_CB = 128
_NCHUNK = _BB // _CB
_CI = _CB * _FIELDS

_mesh = plsc.VectorSubcoreMesh(core_axis_name="c", subcore_axis_name="s")


@functools.partial(
    pl.kernel,
    mesh=_mesh,
    out_type=jax.ShapeDtypeStruct((_FIELDS, 2, _BATCH // 128, 1024),
                                  jnp.float32),
    compiler_params=pltpu.CompilerParams(use_tc_tiling_on_sc=False,
                                         needs_layout_passes=False),
    scratch_types=[
        pltpu.VMEM((_BPW,), jnp.int32),
        pltpu.VMEM((_CI, _D), jnp.float32),
        pltpu.VMEM((_FIELDS * 2048,), jnp.float32),
        pltpu.SemaphoreType.DMA,
        pltpu.SemaphoreType.DMA,
    ],
)
def _embed_gather(idx_hbm, table_hbm, out_hbm, idx_v, rows_v, planes_v,
                  gsem, osem):
    wid = lax.axis_index("s") * 2 + lax.axis_index("c")
    base = wid * _BPW
    pltpu.sync_copy(idx_hbm.at[pl.ds(base, _BPW)], idx_v)

    puts = []
    for c in range(_NCHUNK):
        pltpu.async_copy(
            table_hbm.at[idx_v.at[pl.ds(c * _CI, _CI)]], rows_v, gsem,
        ).wait()

        for h in puts:
            h.wait()
        puts = []
        pbuf = planes_v

        def body(b, _):
            bvec = lax.iota(jnp.int32, 16) * 128 + (
                jnp.full((16,), 0, jnp.int32) + b)
            for f in range(_FIELDS):
                row = rows_v[(b * _FIELDS + f), :]
                plsc.store_scatter(pbuf, [bvec + f * 2048], row)
            return 0

        lax.fori_loop(0, _CB, body, 0)

        tc = wid * _NCHUNK + c
        for f in range(_FIELDS):
            for tr in range(2):
                h = pltpu.async_copy(
                    pbuf.at[pl.ds(f * 2048 + tr * 1024, 1024)],
                    out_hbm.at[f, tr, tc],
                    osem)
                puts.append(h)
    for h in puts:
        h.wait()


def kernel(e, table):
    idx = e.reshape(_B).astype(jnp.int32)
    out6 = _embed_gather(idx, table)
    out5 = out6.reshape(_FIELDS, 2, _BATCH // 128, 8, 128)
    return out5.transpose(2, 4, 0, 1, 3).reshape(_BATCH, _FIELDS, _D)


# speedup vs baseline: 1.6103x; 1.0445x over previous
import functools

import jax
import jax.numpy as jnp
from jax import lax
from jax.experimental import pallas as pl
from jax.experimental.pallas import tpu as pltpu
from jax.experimental.pallas import tpu_sc as plsc

_D = 16
_BATCH = 16384
_FIELDS = 26
_B = _BATCH * _FIELDS
_NW = 32
_BB = _BATCH // _NW
_BPW = _BB * _FIELDS

_mesh = plsc.VectorSubcoreMesh(core_axis_name="c", subcore_axis_name="s")


@functools.partial(
    pl.kernel,
    mesh=_mesh,
    out_type=jax.ShapeDtypeStruct((_FIELDS, 2, _BATCH // 128, 1024),
                                  jnp.float32),
    compiler_params=pltpu.CompilerParams(use_tc_tiling_on_sc=False,
                                         needs_layout_passes=False),
    scratch_types=[
        pltpu.VMEM((_BPW,), jnp.int32),
        pltpu.VMEM((2, _BB, _D), jnp.float32),
        pltpu.VMEM((2, 4 * 2048), jnp.float32),
        pltpu.SemaphoreType.DMA,
        pltpu.SemaphoreType.DMA,
        pltpu.SemaphoreType.DMA,
        pltpu.SemaphoreType.DMA,
        pltpu.SemaphoreType.DMA,
    ],
)
def _embed_gather(eT_hbm, table_hbm, out_hbm, idx_v, rows_v, planes_v,
                  isem, g0, g1, o0, o1):
    wid = lax.axis_index("s") * 2 + lax.axis_index("c")
    b0 = wid * _BB
    gsem = (g0, g1)
    osem = (o0, o1)

    ihs = [
        pltpu.async_copy(eT_hbm.at[f, pl.ds(b0, _BB)],
                         idx_v.at[pl.ds(f * _BB, _BB)], isem)
        for f in range(_FIELDS)
    ]
    for h in ihs:
        h.wait()

    def start_gather(f):
        return pltpu.async_copy(
            table_hbm.at[idx_v.at[pl.ds(f * _BB, _BB)]],
            rows_v.at[f % 2], gsem[f % 2])

    gathers = [None, None]
    puts = [[], []]
    gathers[0] = start_gather(0)
    for f in range(_FIELDS):
        if f + 1 < _FIELDS:
            gathers[(f + 1) % 2] = start_gather(f + 1)
        gathers[f % 2].wait()

        buf = f % 2
        for h in puts[buf]:
            h.wait()
        puts[buf] = []
        pbuf = planes_v.at[buf]
        rbuf = rows_v.at[buf]

        def body(b, _):
            row = rbuf[b, :]
            pos = (lax.shift_right_logical(b, 7) * 2048
                   + lax.bitwise_and(b, 127))
            idx = lax.iota(jnp.int32, 16) * 128 + (
                jnp.full((16,), 0, jnp.int32) + pos)
            plsc.store_scatter(pbuf, [idx], row)
            return 0

        lax.fori_loop(0, _BB, body, 0)

        tc0 = wid * (_BB // 128)
        for q in range(_BB // 128):
            for tr in range(2):
                h = pltpu.async_copy(
                    pbuf.at[pl.ds(q * 2048 + tr * 1024, 1024)],
                    out_hbm.at[f, tr, tc0 + q],
                    osem[buf])
                puts[buf].append(h)
    for hs in puts:
        for h in hs:
            h.wait()


def kernel(e, table):
    out6 = _embed_gather(e.T, table)
    out5 = out6.reshape(_FIELDS, 2, _BATCH // 128, 8, 128)
    return out5.transpose(2, 4, 0, 1, 3).reshape(_BATCH, _FIELDS, _D)


# speedup vs baseline: 1.6105x; 1.0002x over previous
import functools

import jax
import jax.numpy as jnp
from jax import lax
from jax.experimental import pallas as pl
from jax.experimental.pallas import tpu as pltpu
from jax.experimental.pallas import tpu_sc as plsc

_D = 16
_BATCH = 16384
_FIELDS = 26
_B = _BATCH * _FIELDS
_NW = 32
_BB = _BATCH // _NW
_BPW = _BB * _FIELDS

_mesh = plsc.VectorSubcoreMesh(core_axis_name="c", subcore_axis_name="s")


@functools.partial(
    pl.kernel,
    mesh=_mesh,
    out_type=jax.ShapeDtypeStruct((_FIELDS, 2, _BATCH // 128, 1024),
                                  jnp.float32),
    compiler_params=pltpu.CompilerParams(use_tc_tiling_on_sc=False,
                                         needs_layout_passes=False),
    scratch_types=[
        pltpu.VMEM((_BPW,), jnp.int32),

        pltpu.VMEM((2, _BB, _D), jnp.float32),
        pltpu.VMEM((2, 4 * 2048), jnp.float32),
        pltpu.SemaphoreType.DMA,
        pltpu.SemaphoreType.DMA,
        pltpu.SemaphoreType.DMA,
        pltpu.SemaphoreType.DMA,
        pltpu.SemaphoreType.DMA,
    ],
)
def _embed_gather(eflat_hbm, table_hbm, out_hbm, idx_v, rows_v, planes_v,
                  isem, g0, g1, o0, o1):
    wid = lax.axis_index("s") * 2 + lax.axis_index("c")
    b0 = wid * _BB
    gsem = (g0, g1)
    osem = (o0, o1)

    ihs = [
        pltpu.async_copy(eflat_hbm.at[pl.ds(f * _BATCH + b0, _BB)],
                         idx_v.at[pl.ds(f * _BB, _BB)], isem)
        for f in range(_FIELDS)
    ]
    for h in ihs:
        h.wait()

    def start_gather(f):
        return pltpu.async_copy(
            table_hbm.at[idx_v.at[pl.ds(f * _BB, _BB)]],
            rows_v.at[f % 2], gsem[f % 2])

    gathers = [None, None]
    puts = [[], []]
    gathers[0] = start_gather(0)
    for f in range(_FIELDS):
        if f + 1 < _FIELDS:
            gathers[(f + 1) % 2] = start_gather(f + 1)
        gathers[f % 2].wait()

        buf = f % 2
        for h in puts[buf]:
            h.wait()
        puts[buf] = []
        pbuf = planes_v.at[buf]
        rbuf = rows_v.at[buf]

        def body(b, _):
            row = rbuf[b, :]
            pos = (lax.shift_right_logical(b, 7) * 2048
                   + lax.bitwise_and(b, 127))
            idx = lax.iota(jnp.int32, 16) * 128 + (
                jnp.full((16,), 0, jnp.int32) + pos)
            plsc.store_scatter(pbuf, [idx], row)
            return 0

        lax.fori_loop(0, _BB, body, 0)

        tc0 = wid * (_BB // 128)
        for q in range(_BB // 128):
            for tr in range(2):
                h = pltpu.async_copy(
                    pbuf.at[pl.ds(q * 2048 + tr * 1024, 1024)],
                    out_hbm.at[f, tr, tc0 + q],
                    osem[buf])
                puts[buf].append(h)
    for hs in puts:
        for h in hs:
            h.wait()


def kernel(e, table):
    out6 = _embed_gather(e.T.reshape(_B), table)
    out5 = out6.reshape(_FIELDS, 2, _BATCH // 128, 8, 128)
    return out5.transpose(2, 4, 0, 1, 3).reshape(_BATCH, _FIELDS, _D)
